# baseline (speedup 1.0000x reference)
import functools

import jax
import jax.numpy as jnp
from jax import lax
from jax.experimental import pallas as pl
from jax.experimental.pallas import tpu as pltpu
from jax.experimental.pallas import tpu_sc as plsc

_NUM_FIELDS = 26
_VOCAB = 100000
_EMBED_DIM = 32
_BATCH = 4096
_DENSE_NUM = 13
_OUT_DIM = _NUM_FIELDS * _EMBED_DIM + _EMBED_DIM

_NC, _NS, _L = 2, 16, 16
_NW = _NC * _NS
_BPW = _BATCH // _NW


def _make_sc_call():
    mesh = plsc.VectorSubcoreMesh(core_axis_name="c", subcore_axis_name="s")

    @functools.partial(
        pl.kernel,
        mesh=mesh,
        out_type=jax.ShapeDtypeStruct((_BATCH, _OUT_DIM), jnp.float32),
        scratch_types=[
            pltpu.VMEM((_NUM_FIELDS, _BPW), jnp.int32),
            pltpu.VMEM((_BPW, _EMBED_DIM), jnp.float32),
            pltpu.VMEM((_BPW, 16), jnp.float32),
            pltpu.VMEM((_DENSE_NUM, _EMBED_DIM), jnp.float32),
            pltpu.VMEM((_EMBED_DIM,), jnp.float32),
            pltpu.VMEM((_BPW, _EMBED_DIM), jnp.float32),
            pltpu.SemaphoreType.DMA,
        ],
        compiler_params=pltpu.CompilerParams(use_tc_tiling_on_sc=False),
    )
    def sc_embed(table_hbm, idx_hbm, dense_hbm, wt_hbm, b_hbm, out_hbm,
                 idx_v, rows_v, dense_v, wt_v, bias_v, dout_v, sem):
        wid = lax.axis_index("s") * _NC + lax.axis_index("c")
        base = wid * _BPW

        pltpu.sync_copy(idx_hbm.at[:, pl.ds(base, _BPW)], idx_v)
        pltpu.sync_copy(dense_hbm.at[pl.ds(base, _BPW), :], dense_v)
        pltpu.sync_copy(wt_hbm, wt_v)
        pltpu.sync_copy(b_hbm, bias_v)

        def field_body(f, carry):
            pltpu.async_copy(table_hbm.at[idx_v.at[f]], rows_v, sem).wait()
            pltpu.sync_copy(
                rows_v,
                out_hbm.at[pl.ds(base, _BPW), pl.ds(f * _EMBED_DIM, _EMBED_DIM)])
            return carry

        lax.fori_loop(0, _NUM_FIELDS, field_body, 0)

        bias0 = bias_v[pl.ds(0, _L)]
        bias1 = bias_v[pl.ds(_L, _L)]

        def row_body(bb, carry):
            acc0, acc1 = bias0, bias1
            drow = dense_v[bb, pl.ds(0, _L)]
            for kk in range(_DENSE_NUM):
                s = drow[kk]
                acc0 = acc0 + s * wt_v[kk, pl.ds(0, _L)]
                acc1 = acc1 + s * wt_v[kk, pl.ds(_L, _L)]
            dout_v[bb, pl.ds(0, _L)] = acc0
            dout_v[bb, pl.ds(_L, _L)] = acc1
            return carry

        lax.fori_loop(0, _BPW, row_body, 0)
        pltpu.sync_copy(
            dout_v,
            out_hbm.at[pl.ds(base, _BPW),
                       pl.ds(_NUM_FIELDS * _EMBED_DIM, _EMBED_DIM)])

    return sc_embed


_sc_call = _make_sc_call()


def kernel(sparse_indices, dense_features, tables, W, b):
    table_flat = tables.reshape(_NUM_FIELDS * _VOCAB, _EMBED_DIM)
    idx_t = (sparse_indices.astype(jnp.int32)
             + (jnp.arange(_NUM_FIELDS, dtype=jnp.int32) * _VOCAB)[None, :]).T
    dense_pad = jnp.pad(dense_features, ((0, 0), (0, 16 - _DENSE_NUM)))
    return _sc_call(table_flat, idx_t, dense_pad, W.T, b)
